# baseline (speedup 1.0000x reference)
import functools

import jax
import jax.numpy as jnp
from jax import lax
from jax.experimental import pallas as pl
from jax.experimental.pallas import tpu as pltpu
from jax.experimental.pallas import tpu_sc as plsc

BATCH = 16384
K = 16
NUM_CORES = 2
NUM_SUBCORES = 16
NW = NUM_CORES * NUM_SUBCORES
BPW = BATCH // NW
NCHUNK = 4
CHUNK = BPW // NCHUNK
NBLK = BPW // K


def _mf_body(user_hbm, item_hbm, w_hbm, h_hbm, out_hbm,
             uidx_v, iidx_v, u_v, v_v, out_v, sem):
    c = lax.axis_index("c")
    s = lax.axis_index("s")
    wid = s * NUM_CORES + c

    pltpu.sync_copy(user_hbm.at[wid], uidx_v)
    pltpu.sync_copy(item_hbm.at[wid], iidx_v)

    copies = []
    for ci in range(NCHUNK):
        dst = u_v.at[pl.ds(ci * CHUNK, CHUNK), :]
        copies.append(pltpu.async_copy(w_hbm.at[uidx_v.at[ci]], dst, sem))
    for ci in range(NCHUNK):
        dst = v_v.at[pl.ds(ci * CHUNK, CHUNK), :]
        copies.append(pltpu.async_copy(h_hbm.at[iidx_v.at[ci]], dst, sem))
    for cp in copies:
        cp.wait()

    iota = lax.iota(jnp.int32, K)

    def block(blk, _):
        rows = blk * K + iota
        acc = jnp.zeros((K,), jnp.float32)
        for j in range(K):
            cols = jnp.full((K,), j, jnp.int32)
            u = plsc.load_gather(u_v, [rows, cols])
            v = plsc.load_gather(v_v, [rows, cols])
            acc = acc + u * v
        out_v[pl.ds(blk * K, K)] = acc
        return _

    lax.fori_loop(0, NBLK, block, 0)
    pltpu.sync_copy(out_v, out_hbm.at[wid])


@functools.partial(jax.jit, static_argnums=())
def _mf(user, item, w, h):
    mesh = plsc.VectorSubcoreMesh(core_axis_name="c", subcore_axis_name="s")
    f = pl.kernel(
        _mf_body,
        out_type=jax.ShapeDtypeStruct((NW, BPW), jnp.float32),
        mesh=mesh,
        scratch_types=[
            pltpu.VMEM((NCHUNK, CHUNK), jnp.int32),
            pltpu.VMEM((NCHUNK, CHUNK), jnp.int32),
            pltpu.VMEM((BPW, K), jnp.float32),
            pltpu.VMEM((BPW, K), jnp.float32),
            pltpu.VMEM((BPW,), jnp.float32),
            pltpu.SemaphoreType.DMA,
        ],
        compiler_params=pltpu.CompilerParams(
            needs_layout_passes=False, use_tc_tiling_on_sc=False),
    )
    return f(user, item, w, h)


def kernel(x, W, H):
    xi = x.astype(jnp.int32)
    user = xi[:, 0].reshape(NW, NCHUNK, CHUNK)
    item = xi[:, 1].reshape(NW, NCHUNK, CHUNK)
    out = _mf(user, item, W, H)
    return out.reshape(BATCH)


# speedup vs baseline: 1.4563x; 1.4563x over previous
import functools

import jax
import jax.numpy as jnp
from jax import lax
from jax.experimental import pallas as pl
from jax.experimental.pallas import tpu as pltpu
from jax.experimental.pallas import tpu_sc as plsc

BATCH = 16384
K = 16
NUM_CORES = 2
NUM_SUBCORES = 16
NW = NUM_CORES * NUM_SUBCORES
BPW = BATCH // NW
CHUNK = 128
NCHUNK = BPW // CHUNK


def _mf_body(user_hbm, item_hbm, w_hbm, h_hbm, out_hbm,
             uidx_v, iidx_v, u_v, v_v, out_v, sem):
    c = lax.axis_index("c")
    s = lax.axis_index("s")
    wid = s * NUM_CORES + c

    pltpu.sync_copy(user_hbm.at[wid], uidx_v)
    pltpu.sync_copy(item_hbm.at[wid], iidx_v)

    iota = lax.iota(jnp.int32, K)

    def chunk_step(ci, _):
        base = ci * CHUNK
        copies = []
        for g in range(CHUNK // K):
            uu = uidx_v[pl.ds(base + g * K, K)]
            ii = iidx_v[pl.ds(base + g * K, K)]
            for r in range(K):
                copies.append(pltpu.async_copy(
                    w_hbm.at[uu[r]], u_v.at[g * K + r], sem))
                copies.append(pltpu.async_copy(
                    h_hbm.at[ii[r]], v_v.at[g * K + r], sem))
        for cp in copies:
            cp.wait()

        for blk in range(CHUNK // K):
            rows = blk * K + iota
            acc = jnp.zeros((K,), jnp.float32)
            for j in range(K):
                cols = jnp.full((K,), j, jnp.int32)
                u = plsc.load_gather(u_v, [rows, cols])
                v = plsc.load_gather(v_v, [rows, cols])
                acc = acc + u * v
            out_v[pl.ds(base + blk * K, K)] = acc
        return _

    lax.fori_loop(0, NCHUNK, chunk_step, 0)
    pltpu.sync_copy(out_v, out_hbm.at[wid])


@functools.partial(jax.jit, static_argnums=())
def _mf(user, item, w, h):
    mesh = plsc.VectorSubcoreMesh(core_axis_name="c", subcore_axis_name="s")
    f = pl.kernel(
        _mf_body,
        out_type=jax.ShapeDtypeStruct((NW, BPW), jnp.float32),
        mesh=mesh,
        scratch_types=[
            pltpu.VMEM((BPW,), jnp.int32),
            pltpu.VMEM((BPW,), jnp.int32),
            pltpu.VMEM((CHUNK, K), jnp.float32),
            pltpu.VMEM((CHUNK, K), jnp.float32),
            pltpu.VMEM((BPW,), jnp.float32),
            pltpu.SemaphoreType.DMA,
        ],
        compiler_params=pltpu.CompilerParams(
            needs_layout_passes=False, use_tc_tiling_on_sc=True),
    )
    return f(user, item, w, h)


def kernel(x, W, H):
    xi = x.astype(jnp.int32)
    user = xi[:, 0].reshape(NW, BPW)
    item = xi[:, 1].reshape(NW, BPW)
    out = _mf(user, item, W, H)
    return out.reshape(BATCH)


# speedup vs baseline: 6.0719x; 4.1694x over previous
import functools

import jax
import jax.numpy as jnp
from jax import lax
from jax.experimental import pallas as pl
from jax.experimental.pallas import tpu as pltpu
from jax.experimental.pallas import tpu_sc as plsc

BATCH = 16384
K = 16
LANE = 128
NUM_CORES = 2
NUM_SUBCORES = 16
NW = NUM_CORES * NUM_SUBCORES
BPW = BATCH // NW
NGRP = BPW // K


def _mf_body(user_hbm, item_hbm, wt_hbm, ht_hbm, out_hbm,
             uidx_v, iidx_v, uwin_v, vwin_v, out_v, sem):
    c = lax.axis_index("c")
    s = lax.axis_index("s")
    wid = s * NUM_CORES + c

    pltpu.sync_copy(user_hbm.at[wid], uidx_v)
    pltpu.sync_copy(item_hbm.at[wid], iidx_v)

    iota = lax.iota(jnp.int32, K)

    def group_step(g, _):
        uu = uidx_v[pl.ds(g * K, K)]
        ii = iidx_v[pl.ds(g * K, K)]
        ubase = uu & jnp.int32(~(LANE - 1))
        ibase = ii & jnp.int32(~(LANE - 1))
        copies = []
        for r in range(K):
            wu = pl.multiple_of(ubase[r], LANE)
            wi = pl.multiple_of(ibase[r], LANE)
            copies.append(pltpu.async_copy(
                wt_hbm.at[:, pl.ds(wu, LANE)], uwin_v.at[r], sem))
            copies.append(pltpu.async_copy(
                ht_hbm.at[:, pl.ds(wi, LANE)], vwin_v.at[r], sem))
        for cp in copies:
            cp.wait()

        urem = uu & jnp.int32(LANE - 1)
        irem = ii & jnp.int32(LANE - 1)
        acc = jnp.zeros((K,), jnp.float32)
        for j in range(K):
            cols = jnp.full((K,), j, jnp.int32)
            u = plsc.load_gather(uwin_v, [iota, cols, urem])
            v = plsc.load_gather(vwin_v, [iota, cols, irem])
            acc = acc + u * v
        out_v[pl.ds(g * K, K)] = acc
        return _

    lax.fori_loop(0, NGRP, group_step, 0)
    pltpu.sync_copy(out_v, out_hbm.at[wid])


@functools.partial(jax.jit, static_argnums=())
def _mf(user, item, wt, ht):
    mesh = plsc.VectorSubcoreMesh(core_axis_name="c", subcore_axis_name="s")
    f = pl.kernel(
        _mf_body,
        out_type=jax.ShapeDtypeStruct((NW, BPW), jnp.float32),
        mesh=mesh,
        scratch_types=[
            pltpu.VMEM((BPW,), jnp.int32),
            pltpu.VMEM((BPW,), jnp.int32),
            pltpu.VMEM((K, K, LANE), jnp.float32),
            pltpu.VMEM((K, K, LANE), jnp.float32),
            pltpu.VMEM((BPW,), jnp.float32),
            pltpu.SemaphoreType.DMA,
        ],
        compiler_params=pltpu.CompilerParams(
            needs_layout_passes=False, use_tc_tiling_on_sc=True),
    )
    return f(user, item, wt, ht)


def kernel(x, W, H):
    xi = x.astype(jnp.int32)
    user = xi[:, 0].reshape(NW, BPW)
    item = xi[:, 1].reshape(NW, BPW)
    out = _mf(user, item, W.T, H.T)
    return out.reshape(BATCH)


# speedup vs baseline: 7.1472x; 1.1771x over previous
import functools

import jax
import jax.numpy as jnp
from jax import lax
from jax.experimental import pallas as pl
from jax.experimental.pallas import tpu as pltpu
from jax.experimental.pallas import tpu_sc as plsc

BATCH = 16384
K = 16
LANE = 128
NUM_CORES = 2
NUM_SUBCORES = 16
NW = NUM_CORES * NUM_SUBCORES
BPW = BATCH // NW
NGRP = BPW // K


def _mf_body(user_hbm, item_hbm, wt_hbm, ht_hbm, out_hbm,
             uidx_v, iidx_v, win_v, urt_v, vrt_v, out_v, sem):
    c = lax.axis_index("c")
    s = lax.axis_index("s")
    wid = s * NUM_CORES + c

    pltpu.sync_copy(user_hbm.at[wid], uidx_v)
    pltpu.sync_copy(item_hbm.at[wid], iidx_v)

    iota = lax.iota(jnp.int32, K)

    def gather_pass(idx_v, tab_hbm, rt_v):
        def fire(g, b):
            vv = idx_v[pl.ds(g * K, K)] & jnp.int32(~(LANE - 1))
            for r in range(K):
                off = pl.multiple_of(vv[r], LANE)
                pltpu.async_copy(tab_hbm.at[:, pl.ds(off, LANE)],
                                 win_v.at[b, r], sem)

        fire(0, 0)

        def group_step(g, _):
            b = lax.rem(g, 2)

            @pl.when(g + 1 < NGRP)
            def _fire_next():
                fire(g + 1, lax.rem(g + 1, 2))

            for r in range(K):
                pltpu.make_async_copy(tab_hbm.at[:, pl.ds(0, LANE)],
                                      win_v.at[b, r], sem).wait()

            rem = idx_v[pl.ds(g * K, K)] & jnp.int32(LANE - 1)
            bvec = jnp.full((K,), b, jnp.int32)
            for j in range(K):
                cols = jnp.full((K,), j, jnp.int32)
                vals = plsc.load_gather(win_v, [bvec, iota, cols, rem])
                rt_v[j, pl.ds(g * K, K)] = vals
            return _

        lax.fori_loop(0, NGRP, group_step, 0)

    gather_pass(uidx_v, wt_hbm, urt_v)
    gather_pass(iidx_v, ht_hbm, vrt_v)

    def dot_step(g, _):
        acc = jnp.zeros((K,), jnp.float32)
        for j in range(K):
            acc = acc + urt_v[j, pl.ds(g * K, K)] * vrt_v[j, pl.ds(g * K, K)]
        out_v[pl.ds(g * K, K)] = acc
        return _

    lax.fori_loop(0, NGRP, dot_step, 0)
    pltpu.sync_copy(out_v, out_hbm.at[wid])


@functools.partial(jax.jit, static_argnums=())
def _mf(user, item, wt, ht):
    mesh = plsc.VectorSubcoreMesh(core_axis_name="c", subcore_axis_name="s")
    f = pl.kernel(
        _mf_body,
        out_type=jax.ShapeDtypeStruct((NW, BPW), jnp.float32),
        mesh=mesh,
        scratch_types=[
            pltpu.VMEM((BPW,), jnp.int32),
            pltpu.VMEM((BPW,), jnp.int32),
            pltpu.VMEM((2, K, K, LANE), jnp.float32),
            pltpu.VMEM((K, BPW), jnp.float32),
            pltpu.VMEM((K, BPW), jnp.float32),
            pltpu.VMEM((BPW,), jnp.float32),
            pltpu.SemaphoreType.DMA,
        ],
        compiler_params=pltpu.CompilerParams(
            needs_layout_passes=False, use_tc_tiling_on_sc=True),
    )
    return f(user, item, wt, ht)


def kernel(x, W, H):
    xi = x.astype(jnp.int32)
    user = xi[:, 0].reshape(NW, BPW)
    item = xi[:, 1].reshape(NW, BPW)
    out = _mf(user, item, W.T, H.T)
    return out.reshape(BATCH)
